# Initial kernel scaffold; baseline (speedup 1.0000x reference)
import jax
import jax.numpy as jnp
from jax.experimental import pallas as pl


def kernel(x, edge_index, edge_attr, batch, Wq0, bq0, Wk0, bk0, Wv0, bv0, We0, Ws0, bs0, g0, be0, Wq1, bq1, Wk1, bk1, Wv1, bv1, We1, Ws1, bs1, g1, be1):
    raise NotImplementedError("write your pallas kernel here")


# speedup vs baseline: 1.4625x; 1.4625x over previous
import jax
import jax.numpy as jnp
from jax.experimental import pallas as pl


def _affine_pallas(x, g, b):
    def body(x_ref, g_ref, b_ref, o_ref):
        o_ref[...] = x_ref[...] * g_ref[...] + b_ref[...]
    return pl.pallas_call(
        body,
        out_shape=jax.ShapeDtypeStruct(x.shape, x.dtype),
    )(x, g[None, :], b[None, :])


def _conv(x, src, dst, a, Wq, bq, Wk, bk, Wv, bv, We, Ws, bs):
    n = x.shape[0]
    q = x @ Wq + bq
    k = x @ Wk + bk
    v = x @ Wv + bv
    u = q @ We[0]
    d = q.shape[-1]
    inv = 1.0 / jnp.sqrt(jnp.float32(d))
    s = jnp.sum(q[dst] * k[src], axis=-1) + a * u[dst]
    ex = jnp.exp(s * inv)
    den = jax.ops.segment_sum(ex, dst, num_segments=n)
    num = jax.ops.segment_sum(ex[:, None] * v[src], dst, num_segments=n)
    w = jax.ops.segment_sum(ex * a, dst, num_segments=n)
    den = jnp.maximum(den, 1e-16)
    out = num / den[:, None] + (w / den)[:, None] * We[0][None, :]
    return out + (x @ Ws + bs)


def _bn(x, gamma, beta, eps=1e-5):
    mu = jnp.mean(x, axis=0)
    var = jnp.var(x, axis=0)
    return _affine_pallas((x - mu) / jnp.sqrt(var + eps), gamma, beta)


def _pool(x, batch, num_graphs):
    s = jax.ops.segment_sum(x, batch, num_segments=num_graphs)
    cnt = jax.ops.segment_sum(jnp.ones((x.shape[0],), dtype=x.dtype), batch, num_segments=num_graphs)
    return s / jnp.maximum(cnt, 1.0)[:, None]


def kernel(x, edge_index, edge_attr, batch, Wq0, bq0, Wk0, bk0, Wv0, bv0, We0, Ws0, bs0, g0, be0, Wq1, bq1, Wk1, bk1, Wv1, bv1, We1, Ws1, bs1, g1, be1):
    src = edge_index[0]
    dst = edge_index[1]
    a = edge_attr[:, 0]
    G = 64
    h0 = _conv(x, src, dst, a, Wq0, bq0, Wk0, bk0, Wv0, bv0, We0, Ws0, bs0)
    h0 = _bn(h0, g0, be0)
    h1 = _conv(h0, src, dst, a, Wq1, bq1, Wk1, bk1, Wv1, bv1, We1, Ws1, bs1)
    h1 = _bn(h1, g1, be1)
    xs = [h0, h1]
    xpool = [_pool(t, batch, G) for t in xs]
    return (jnp.concatenate(xpool, 1), jnp.concatenate(xs, 1))


# speedup vs baseline: 3.3029x; 2.2583x over previous
import dataclasses
import functools
import math

import jax
import jax.numpy as jnp
from jax import lax
from jax.experimental import pallas as pl
from jax.experimental.pallas import tpu as pltpu
from jax.experimental.pallas import tpu_sc as plsc

N = 10000
E = 320000
F_IN = 128
DIM = 32
HID = 128
G = 64

NP = 10240
NACC = 10112
STRIPE = NACC // 16
C = 128
NCH_RAW = E // C
NCH = 2528
EP = NCH * C

_HIGH = lax.Precision.HIGHEST


def _dot(a, b):
    return lax.dot_general(a, b, (((1,), (0,)), ((), ())),
                           precision=_HIGH, preferred_element_type=jnp.float32)


def _dense0_body(x_ref, wq_ref, bq_ref, wk_ref, bk_ref, wv_ref, bv_ref,
                 we_ref, ws_ref, bs_ref,
                 q_ref, k_ref, va_ref, vb_ref, vc_ref, vd_ref, s_ref, u_ref):
    x = x_ref[...]
    q = _dot(x, wq_ref[...]) + bq_ref[...][None, :]
    q_ref[...] = q
    k_ref[...] = _dot(x, wk_ref[...]) + bk_ref[...][None, :]
    v = _dot(x, wv_ref[...]) + bv_ref[...][None, :]
    va_ref[...] = v[:, 0:HID // 4]
    vb_ref[...] = v[:, HID // 4:HID // 2]
    vc_ref[...] = v[:, HID // 2:3 * HID // 4]
    vd_ref[...] = v[:, 3 * HID // 4:HID]
    s_ref[...] = _dot(x, ws_ref[...]) + bs_ref[...][None, :]
    u2 = lax.dot_general(q, we_ref[...], (((1,), (1,)), ((), ())),
                         precision=_HIGH, preferred_element_type=jnp.float32)
    u_ref[...] = u2[:, 0]


def _dense0(x_pad, Wq, bq, Wk, bk, Wv, bv, We, Ws, bs):
    d = Wq.shape[1]
    blk = 1024
    grid = NP // blk
    full2 = lambda a: pl.BlockSpec(a.shape, lambda i: (0,) * a.ndim)
    row = pl.BlockSpec((blk, F_IN), lambda i: (i, 0))
    rowo = pl.BlockSpec((blk, d), lambda i: (i, 0))
    rowh = pl.BlockSpec((blk, d // 4), lambda i: (i, 0))
    return pl.pallas_call(
        _dense0_body,
        grid=(grid,),
        in_specs=[row, full2(Wq), full2(bq), full2(Wk), full2(bk),
                  full2(Wv), full2(bv), full2(We), full2(Ws), full2(bs)],
        out_specs=[rowo, rowo, rowh, rowh, rowh, rowh, rowo,
                   pl.BlockSpec((blk,), lambda i: (i,))],
        out_shape=[jax.ShapeDtypeStruct((NP, d), jnp.float32)] * 2 +
                  [jax.ShapeDtypeStruct((NP, d // 4), jnp.float32)] * 4 +
                  [jax.ShapeDtypeStruct((NP, d), jnp.float32),
                   jax.ShapeDtypeStruct((NP,), jnp.float32)],
    )(x_pad, Wq, bq, Wk, bk, Wv, bv, We, Ws, bs)


def _make_sc_pass(d, num_cores):
    RW = d + 16
    QT = d // 16
    tiles = 16 * num_cores
    ch_per_tile = NCH // tiles
    inv = 1.0 / math.sqrt(d)
    mesh = plsc.VectorSubcoreMesh(core_axis_name="c", subcore_axis_name="s",
                                  num_cores=num_cores)
    cp = pltpu.CompilerParams()
    if "needs_layout_passes" in pltpu.CompilerParams.__dataclass_fields__:
        cp = dataclasses.replace(cp, needs_layout_passes=False,
                                 use_tc_tiling_on_sc=False)

    @functools.partial(
        pl.kernel,
        out_type=jax.ShapeDtypeStruct((num_cores, NACC, RW), jnp.float32),
        mesh=mesh,
        compiler_params=cp,
        scratch_types=[
            pltpu.VMEM((NP,), jnp.float32),
            pltpu.VMEM((C,), jnp.int32),
            pltpu.VMEM((C,), jnp.int32),
            pltpu.VMEM((C,), jnp.float32),
            pltpu.VMEM((C, d), jnp.float32),
            pltpu.VMEM((C, d), jnp.float32),
            pltpu.VMEM((C, d), jnp.float32),
            pltpu.VMEM((C, RW), jnp.float32),
            pltpu.VMEM((C,), jnp.float32),
            pltpu.VMEM((C + 16,), jnp.float32),
            pltpu.VMEM((C + 16,), jnp.float32),
            pltpu.VMEM_SHARED((NACC, RW), jnp.float32),
        ],
    )
    def sc_pass(q_hbm, k_hbm, v_hbm, u_hbm, src_hbm, dst_hbm, a_hbm, z_hbm,
                out_hbm, u_v, src_v, dst_v, a_v, q_v, k_v, v_v, r_v,
                s_v, ex_v, exa_v, acc_s):
        cid = lax.axis_index("c")
        sid = lax.axis_index("s")
        wid = sid * num_cores + cid

        pltpu.sync_copy(u_hbm, u_v)
        pltpu.sync_copy(z_hbm, acc_s.at[pl.ds(pl.multiple_of(sid * STRIPE, 8), STRIPE)])
        plsc.subcore_barrier()

        i16 = lax.iota(jnp.int32, 16)
        m0 = i16 == 0
        m1 = i16 == 1

        @pl.loop(0, ch_per_tile)
        def _chunk(t):
            ch = wid * ch_per_tile + t
            pltpu.sync_copy(src_hbm.at[ch], src_v)
            pltpu.sync_copy(dst_hbm.at[ch], dst_v)
            pltpu.sync_copy(a_hbm.at[ch], a_v)
            pltpu.sync_copy(k_hbm.at[src_v], k_v)
            pltpu.sync_copy(v_hbm.at[src_v], v_v)
            pltpu.sync_copy(q_hbm.at[dst_v], q_v)

            @pl.loop(0, C)
            def _dotloop(e):
                acc = q_v[e, pl.ds(0, 16)] * k_v[e, pl.ds(0, 16)]
                for j in range(1, QT):
                    acc = acc + q_v[e, pl.ds(16 * j, 16)] * k_v[e, pl.ds(16 * j, 16)]
                tot = jnp.sum(acc)
                base = (e // 16) * 16
                sl = pl.ds(base, 16)
                old = s_v[sl]
                s_v[sl] = jnp.where(i16 == (e - base), jnp.full((16,), tot),
                                    old)

            for g in range(C // 16):
                sl = pl.ds(g * 16, 16)
                dst16 = dst_v[sl]
                u16 = plsc.load_gather(u_v, [dst16])
                a16 = a_v[sl]
                ex16 = jnp.exp((s_v[sl] + a16 * u16) * inv)
                ex_v[sl] = ex16
                exa_v[sl] = ex16 * a16

            @pl.loop(0, C)
            def _rowloop(e):
                ex_bc = jnp.full((16,), ex_v[pl.ds(e, 16)][0])
                exa_bc = jnp.full((16,), exa_v[pl.ds(e, 16)][0])
                r_v[e, pl.ds(d, 16)] = (jnp.where(m0, ex_bc, 0.0) +
                                        jnp.where(m1, exa_bc, 0.0))
                for j in range(QT):
                    r_v[e, pl.ds(16 * j, 16)] = v_v[e, pl.ds(16 * j, 16)] * ex_bc

            pltpu.sync_copy(r_v, acc_s.at[dst_v], add=True)

        plsc.subcore_barrier()
        pltpu.sync_copy(acc_s.at[pl.ds(pl.multiple_of(sid * STRIPE, 8), STRIPE)],
                        out_hbm.at[cid, pl.ds(pl.multiple_of(sid * STRIPE, 8),
                                              STRIPE)])

    return sc_pass


_sc_pass1 = _make_sc_pass(DIM, num_cores=1)


def _make_sc_pass0():
    d = HID
    H = d // 4
    RW = H + 16
    tiles = 16
    cpt = NCH // tiles
    EPT = cpt * C
    inv = 1.0 / math.sqrt(d)
    mesh = plsc.VectorSubcoreMesh(core_axis_name="c", subcore_axis_name="s",
                                  num_cores=1)
    cp = pltpu.CompilerParams()
    if "needs_layout_passes" in pltpu.CompilerParams.__dataclass_fields__:
        cp = dataclasses.replace(cp, needs_layout_passes=False,
                                 use_tc_tiling_on_sc=False)

    @functools.partial(
        pl.kernel,
        out_type=jax.ShapeDtypeStruct((4, NACC, RW), jnp.float32),
        mesh=mesh,
        compiler_params=cp,
        scratch_types=[
            pltpu.VMEM((NP,), jnp.float32),
            pltpu.VMEM((C,), jnp.int32),
            pltpu.VMEM((C,), jnp.int32),
            pltpu.VMEM((C,), jnp.float32),
            pltpu.VMEM((C, d), jnp.float32),
            pltpu.VMEM((C, d), jnp.float32),
            pltpu.VMEM((C, H), jnp.float32),
            pltpu.VMEM((C, RW), jnp.float32),
            pltpu.VMEM((C,), jnp.float32),
            pltpu.VMEM((EPT + 16,), jnp.float32),
            pltpu.VMEM((EPT + 16,), jnp.float32),
            pltpu.VMEM_SHARED((NACC, RW), jnp.float32),
        ],
    )
    def sc_pass0(q_hbm, k_hbm, va_hbm, vb_hbm, vc_hbm, vd_hbm, u_hbm,
                 src_hbm, dst_hbm, a_hbm, z_hbm, out_hbm, u_v, src_v, dst_v,
                 a_v, q_v, k_v, v_v, r_v, s_v, ex_v, exa_v, acc_s):
        sid = lax.axis_index("s")
        wid = sid

        pltpu.sync_copy(u_hbm, u_v)
        stripe = pl.ds(pl.multiple_of(sid * STRIPE, 8), STRIPE)
        pltpu.sync_copy(z_hbm, acc_s.at[stripe])
        plsc.subcore_barrier()

        i16 = lax.iota(jnp.int32, 16)
        m0 = i16 == 0
        m1 = i16 == 1

        @pl.loop(0, cpt)
        def _chunk(t):
            ch = wid * cpt + t
            off = t * C
            pltpu.sync_copy(src_hbm.at[ch], src_v)
            pltpu.sync_copy(dst_hbm.at[ch], dst_v)
            pltpu.sync_copy(a_hbm.at[ch], a_v)
            pltpu.sync_copy(k_hbm.at[src_v], k_v)
            pltpu.sync_copy(va_hbm.at[src_v], v_v)
            pltpu.sync_copy(q_hbm.at[dst_v], q_v)

            @pl.loop(0, C)
            def _dotloop(e):
                acc = q_v[e, pl.ds(0, 16)] * k_v[e, pl.ds(0, 16)]
                for j in range(1, d // 16):
                    acc = acc + q_v[e, pl.ds(16 * j, 16)] * k_v[e, pl.ds(16 * j, 16)]
                tot = jnp.sum(acc)
                base = (e // 16) * 16
                sl = pl.ds(base, 16)
                old = s_v[sl]
                s_v[sl] = jnp.where(i16 == (e - base), jnp.full((16,), tot),
                                    old)

            for g in range(C // 16):
                sl = pl.ds(g * 16, 16)
                osl = pl.ds(off + g * 16, 16)
                dst16 = dst_v[sl]
                u16 = plsc.load_gather(u_v, [dst16])
                a16 = a_v[sl]
                ex16 = jnp.exp((s_v[sl] + a16 * u16) * inv)
                ex_v[osl] = ex16
                exa_v[osl] = ex16 * a16

            @pl.loop(0, C)
            def _rowloop(e):
                ex_bc = jnp.full((16,), ex_v[pl.ds(off + e, 16)][0])
                exa_bc = jnp.full((16,), exa_v[pl.ds(off + e, 16)][0])
                r_v[e, pl.ds(H, 16)] = (jnp.where(m0, ex_bc, 0.0) +
                                        jnp.where(m1, exa_bc, 0.0))
                for j in range(H // 16):
                    r_v[e, pl.ds(16 * j, 16)] = v_v[e, pl.ds(16 * j, 16)] * ex_bc

            pltpu.sync_copy(r_v, acc_s.at[dst_v], add=True)

        plsc.subcore_barrier()
        pltpu.sync_copy(acc_s.at[stripe], out_hbm.at[0, stripe])

        z16 = jnp.zeros((16,), jnp.float32)
        for p, vp_hbm in enumerate([vb_hbm, vc_hbm, vd_hbm], start=1):
            plsc.subcore_barrier()
            pltpu.sync_copy(z_hbm, acc_s.at[stripe])
            plsc.subcore_barrier()
            if p == 1:
                @pl.loop(0, C)
                def _zerotail(e):
                    r_v[e, pl.ds(H, 16)] = z16

            @pl.loop(0, cpt)
            def _chunk_b(t):
                ch = wid * cpt + t
                off = t * C
                pltpu.sync_copy(dst_hbm.at[ch], dst_v)
                pltpu.sync_copy(src_hbm.at[ch], src_v)
                pltpu.sync_copy(vp_hbm.at[src_v], v_v)

                @pl.loop(0, C)
                def _rowloop_b(e):
                    ex_bc = jnp.full((16,), ex_v[pl.ds(off + e, 16)][0])
                    for j in range(H // 16):
                        r_v[e, pl.ds(16 * j, 16)] = (
                            v_v[e, pl.ds(16 * j, 16)] * ex_bc)

                pltpu.sync_copy(r_v, acc_s.at[dst_v], add=True)

            plsc.subcore_barrier()
            pltpu.sync_copy(acc_s.at[stripe], out_hbm.at[p, stripe])

    return sc_pass0


_sc_pass0 = _make_sc_pass0()


def _combine0_body(acc_ref, skip_ref, we0_ref, g0_ref, be0_ref, h0f_ref):
    H = HID // 4
    num = jnp.concatenate([acc_ref[p, 0:N, 0:H] for p in range(4)], axis=1)
    den = jnp.maximum(acc_ref[0, 0:N, H:H + 1], 1e-16)
    w = acc_ref[0, 0:N, H + 1:H + 2]
    out0 = num / den + (w / den) * we0_ref[...] + skip_ref[0:N, :]
    mu = jnp.mean(out0, axis=0, keepdims=True)
    var = jnp.mean(out0 * out0, axis=0, keepdims=True) - mu * mu
    h0 = (out0 - mu) / jnp.sqrt(var + 1e-5) * g0_ref[...][None, :] + \
        be0_ref[...][None, :]
    h0f_ref[0:N, :] = h0
    h0f_ref[N:NP, :] = jnp.zeros((NP - N, HID), jnp.float32)


def _combine0(acc, skip0, We0, g0, be0):
    return pl.pallas_call(
        _combine0_body,
        out_shape=jax.ShapeDtypeStruct((NP, HID), jnp.float32),
    )(acc, skip0, We0, g0, be0)


def _dense1_body(x_ref, wq_ref, bq_ref, wk_ref, bk_ref, wv_ref, bv_ref,
                 we_ref, ws_ref, bs_ref,
                 q_ref, k_ref, v_ref, s_ref, u_ref):
    x = x_ref[...]
    q = _dot(x, wq_ref[...]) + bq_ref[...][None, :]
    q_ref[...] = q
    k_ref[...] = _dot(x, wk_ref[...]) + bk_ref[...][None, :]
    v_ref[...] = _dot(x, wv_ref[...]) + bv_ref[...][None, :]
    s_ref[...] = _dot(x, ws_ref[...]) + bs_ref[...][None, :]
    u2 = lax.dot_general(q, we_ref[...], (((1,), (1,)), ((), ())),
                         precision=_HIGH, preferred_element_type=jnp.float32)
    u_ref[...] = u2[:, 0]


def _dense1(h0f, Wq, bq, Wk, bk, Wv, bv, We, Ws, bs):
    d = Wq.shape[1]
    blk = 1024
    grid = NP // blk
    full2 = lambda a: pl.BlockSpec(a.shape, lambda i: (0,) * a.ndim)
    row = pl.BlockSpec((blk, HID), lambda i: (i, 0))
    rowo = pl.BlockSpec((blk, d), lambda i: (i, 0))
    return pl.pallas_call(
        _dense1_body,
        grid=(grid,),
        in_specs=[row, full2(Wq), full2(bq), full2(Wk), full2(bk),
                  full2(Wv), full2(bv), full2(We), full2(Ws), full2(bs)],
        out_specs=[rowo, rowo, rowo, rowo,
                   pl.BlockSpec((blk,), lambda i: (i,))],
        out_shape=[jax.ShapeDtypeStruct((NP, d), jnp.float32)] * 4 +
                  [jax.ShapeDtypeStruct((NP,), jnp.float32)],
    )(h0f, Wq, bq, Wk, bk, Wv, bv, We, Ws, bs)


def _combine1_body(acc_ref, skip_ref, h0f_ref, batch_ref, we1_ref,
                   g1_ref, be1_ref, pool_ref, xs_ref):
    num = acc_ref[0, 0:N, 0:DIM]
    den = jnp.maximum(acc_ref[0, 0:N, DIM:DIM + 1], 1e-16)
    w = acc_ref[0, 0:N, DIM + 1:DIM + 2]
    out1 = num / den + (w / den) * we1_ref[...] + skip_ref[0:N, :]
    mu = jnp.mean(out1, axis=0, keepdims=True)
    var = jnp.mean(out1 * out1, axis=0, keepdims=True) - mu * mu
    h1 = (out1 - mu) / jnp.sqrt(var + 1e-5) * g1_ref[...][None, :] + \
        be1_ref[...][None, :]

    xs_ref[:, 0:HID] = h0f_ref[0:N, :]
    xs_ref[:, HID:HID + DIM] = h1

    b = batch_ref[...]
    onehot = (lax.broadcasted_iota(jnp.int32, (G, NP), 0) ==
              b[None, :]).astype(jnp.float32)
    xs_pad = jnp.concatenate(
        [h0f_ref[...],
         jnp.concatenate([h1, jnp.zeros((NP - N, DIM), jnp.float32)], axis=0)],
        axis=1)
    sums = _dot(onehot, xs_pad)
    cnt = jnp.sum(onehot, axis=1, keepdims=True)
    pool_ref[...] = sums / jnp.maximum(cnt, 1.0)


def _combine1(acc, skip1, h0f, batch_pad, We1, g1, be1):
    return pl.pallas_call(
        _combine1_body,
        out_shape=[jax.ShapeDtypeStruct((G, HID + DIM), jnp.float32),
                   jax.ShapeDtypeStruct((N, HID + DIM), jnp.float32)],
    )(acc, skip1, h0f, batch_pad, We1, g1, be1)


def _stage_body(ei_ref, ea_ref, src_ref, dst_ref, a_ref):
    src_ref[0:NCH_RAW, :] = ei_ref[0, :].reshape(NCH_RAW, C)
    src_ref[NCH_RAW:NCH, :] = jnp.zeros((NCH - NCH_RAW, C), jnp.int32)
    dst_ref[0:NCH_RAW, :] = ei_ref[1, :].reshape(NCH_RAW, C)
    dst_ref[NCH_RAW:NCH, :] = jnp.full((NCH - NCH_RAW, C), N, jnp.int32)
    a_ref[0:NCH_RAW, :] = ea_ref[:, 0].reshape(NCH_RAW, C)
    a_ref[NCH_RAW:NCH, :] = jnp.zeros((NCH - NCH_RAW, C), jnp.float32)


def _stage_edges(edge_index, edge_attr):
    return pl.pallas_call(
        _stage_body,
        out_shape=[jax.ShapeDtypeStruct((NCH, C), jnp.int32),
                   jax.ShapeDtypeStruct((NCH, C), jnp.int32),
                   jax.ShapeDtypeStruct((NCH, C), jnp.float32)],
    )(edge_index, edge_attr)


def kernel(x, edge_index, edge_attr, batch, Wq0, bq0, Wk0, bk0, Wv0, bv0,
           We0, Ws0, bs0, g0, be0, Wq1, bq1, Wk1, bk1, Wv1, bv1, We1, Ws1,
           bs1, g1, be1):
    x_pad = jnp.pad(x, ((0, NP - N), (0, 0)))
    src = edge_index[0]
    dst = edge_index[1]
    a = edge_attr[:, 0]
    srcp = jnp.concatenate([src, jnp.zeros((EP - E,), jnp.int32)]).reshape(NCH, C)
    dstp = jnp.concatenate([dst, jnp.full((EP - E,), N, jnp.int32)]).reshape(NCH, C)
    ap = jnp.concatenate([a, jnp.zeros((EP - E,), jnp.float32)]).reshape(NCH, C)
    batch_pad = jnp.concatenate([batch, jnp.full((NP - N,), G, jnp.int32)])
    z0 = jnp.zeros((STRIPE, HID // 4 + 16), jnp.float32)
    z1 = jnp.zeros((STRIPE, DIM + 16), jnp.float32)

    q0, k0, va0, vb0, vc0, vd0, skip0, u0 = _dense0(
        x_pad, Wq0, bq0, Wk0, bk0, Wv0, bv0, We0, Ws0, bs0)
    acc0 = _sc_pass0(q0, k0, va0, vb0, vc0, vd0, u0, srcp, dstp, ap, z0)
    h0f = _combine0(acc0, skip0, We0, g0, be0)
    q1, k1, v1, skip1, u1 = _dense1(h0f, Wq1, bq1, Wk1, bk1, Wv1, bv1,
                                    We1, Ws1, bs1)

    acc1 = _sc_pass1(q1, k1, v1, u1, srcp, dstp, ap, z1)
    pooled, xs = _combine1(acc1, skip1, h0f, batch_pad, We1, g1, be1)
    return (pooled, xs)


# speedup vs baseline: 5.5650x; 1.6849x over previous
import dataclasses
import functools
import math

import jax
import jax.numpy as jnp
from jax import lax
from jax.experimental import pallas as pl
from jax.experimental.pallas import tpu as pltpu
from jax.experimental.pallas import tpu_sc as plsc

N = 10000
E = 320000
F_IN = 128
DIM = 32
HID = 128
G = 64

NP = 10240
NACC = 10112
STRIPE = NACC // 16
C = 128
NCH_RAW = E // C
NCH = 2528
EP = NCH * C

_HIGH = lax.Precision.HIGHEST


def _dot(a, b):
    return lax.dot_general(a, b, (((1,), (0,)), ((), ())),
                           precision=_HIGH, preferred_element_type=jnp.float32)


def _dense0_body(x_ref, wq_ref, bq_ref, wk_ref, bk_ref, wv_ref, bv_ref,
                 we_ref, ws_ref, bs_ref,
                 q_ref, k_ref, va_ref, vb_ref, vc_ref, vd_ref, s_ref, u_ref):
    x = x_ref[...]
    q = _dot(x, wq_ref[...]) + bq_ref[...][None, :]
    q_ref[...] = q
    k_ref[...] = _dot(x, wk_ref[...]) + bk_ref[...][None, :]
    v = _dot(x, wv_ref[...]) + bv_ref[...][None, :]
    va_ref[...] = v[:, 0:HID // 4]
    vb_ref[...] = v[:, HID // 4:HID // 2]
    vc_ref[...] = v[:, HID // 2:3 * HID // 4]
    vd_ref[...] = v[:, 3 * HID // 4:HID]
    s_ref[...] = _dot(x, ws_ref[...]) + bs_ref[...][None, :]
    u2 = lax.dot_general(q, we_ref[...], (((1,), (1,)), ((), ())),
                         precision=_HIGH, preferred_element_type=jnp.float32)
    u_ref[...] = u2[:, 0]


def _dense0(x_pad, Wq, bq, Wk, bk, Wv, bv, We, Ws, bs):
    d = Wq.shape[1]
    blk = 1024
    grid = NP // blk
    full2 = lambda a: pl.BlockSpec(a.shape, lambda i: (0,) * a.ndim)
    row = pl.BlockSpec((blk, F_IN), lambda i: (i, 0))
    rowo = pl.BlockSpec((blk, d), lambda i: (i, 0))
    rowh = pl.BlockSpec((blk, d // 4), lambda i: (i, 0))
    return pl.pallas_call(
        _dense0_body,
        grid=(grid,),
        in_specs=[row, full2(Wq), full2(bq), full2(Wk), full2(bk),
                  full2(Wv), full2(bv), full2(We), full2(Ws), full2(bs)],
        out_specs=[rowo, rowo, rowh, rowh, rowh, rowh, rowo,
                   pl.BlockSpec((blk,), lambda i: (i,))],
        out_shape=[jax.ShapeDtypeStruct((NP, d), jnp.float32)] * 2 +
                  [jax.ShapeDtypeStruct((NP, d // 4), jnp.float32)] * 4 +
                  [jax.ShapeDtypeStruct((NP, d), jnp.float32),
                   jax.ShapeDtypeStruct((NP,), jnp.float32)],
    )(x_pad, Wq, bq, Wk, bk, Wv, bv, We, Ws, bs)


def _make_sc_pass(d, num_cores):
    RW = d + 16
    QT = d // 16
    tiles = 16 * num_cores
    ch_per_tile = NCH // tiles
    inv = 1.0 / math.sqrt(d)
    mesh = plsc.VectorSubcoreMesh(core_axis_name="c", subcore_axis_name="s",
                                  num_cores=num_cores)
    cp = pltpu.CompilerParams()
    if "needs_layout_passes" in pltpu.CompilerParams.__dataclass_fields__:
        cp = dataclasses.replace(cp, needs_layout_passes=False,
                                 use_tc_tiling_on_sc=False)

    @functools.partial(
        pl.kernel,
        out_type=jax.ShapeDtypeStruct((num_cores, NACC, RW), jnp.float32),
        mesh=mesh,
        compiler_params=cp,
        scratch_types=[
            pltpu.VMEM((NP,), jnp.float32),
            pltpu.VMEM((C,), jnp.int32),
            pltpu.VMEM((C,), jnp.int32),
            pltpu.VMEM((C,), jnp.float32),
            pltpu.VMEM((C, d), jnp.float32),
            pltpu.VMEM((C, d), jnp.float32),
            pltpu.VMEM((C, d), jnp.float32),
            pltpu.VMEM((C, RW), jnp.float32),
            pltpu.VMEM((C,), jnp.float32),
            pltpu.VMEM((C + 16,), jnp.float32),
            pltpu.VMEM((C + 16,), jnp.float32),
            pltpu.VMEM_SHARED((NACC, RW), jnp.float32),
        ],
    )
    def sc_pass(q_hbm, k_hbm, v_hbm, u_hbm, src_hbm, dst_hbm, a_hbm, z_hbm,
                out_hbm, u_v, src_v, dst_v, a_v, q_v, k_v, v_v, r_v,
                s_v, ex_v, exa_v, acc_s):
        cid = lax.axis_index("c")
        sid = lax.axis_index("s")
        wid = sid * num_cores + cid

        pltpu.sync_copy(u_hbm, u_v)
        pltpu.sync_copy(z_hbm, acc_s.at[pl.ds(pl.multiple_of(sid * STRIPE, 8), STRIPE)])
        plsc.subcore_barrier()

        i16 = lax.iota(jnp.int32, 16)
        m0 = i16 == 0
        m1 = i16 == 1

        @pl.loop(0, ch_per_tile)
        def _chunk(t):
            ch = wid * ch_per_tile + t
            pltpu.sync_copy(src_hbm.at[ch], src_v)
            pltpu.sync_copy(dst_hbm.at[ch], dst_v)
            pltpu.sync_copy(a_hbm.at[ch], a_v)
            pltpu.sync_copy(k_hbm.at[src_v], k_v)
            pltpu.sync_copy(v_hbm.at[src_v], v_v)
            pltpu.sync_copy(q_hbm.at[dst_v], q_v)

            @pl.loop(0, C)
            def _dotloop(e):
                acc = q_v[e, pl.ds(0, 16)] * k_v[e, pl.ds(0, 16)]
                for j in range(1, QT):
                    acc = acc + q_v[e, pl.ds(16 * j, 16)] * k_v[e, pl.ds(16 * j, 16)]
                tot = jnp.sum(acc)
                base = (e // 16) * 16
                sl = pl.ds(base, 16)
                old = s_v[sl]
                s_v[sl] = jnp.where(i16 == (e - base), jnp.full((16,), tot),
                                    old)

            for g in range(C // 16):
                sl = pl.ds(g * 16, 16)
                dst16 = dst_v[sl]
                u16 = plsc.load_gather(u_v, [dst16])
                a16 = a_v[sl]
                ex16 = jnp.exp((s_v[sl] + a16 * u16) * inv)
                ex_v[sl] = ex16
                exa_v[sl] = ex16 * a16

            @pl.loop(0, C)
            def _rowloop(e):
                ex_bc = jnp.full((16,), ex_v[pl.ds(e, 16)][0])
                exa_bc = jnp.full((16,), exa_v[pl.ds(e, 16)][0])
                r_v[e, pl.ds(d, 16)] = (jnp.where(m0, ex_bc, 0.0) +
                                        jnp.where(m1, exa_bc, 0.0))
                for j in range(QT):
                    r_v[e, pl.ds(16 * j, 16)] = v_v[e, pl.ds(16 * j, 16)] * ex_bc

            pltpu.sync_copy(r_v, acc_s.at[dst_v], add=True)

        plsc.subcore_barrier()
        pltpu.sync_copy(acc_s.at[pl.ds(pl.multiple_of(sid * STRIPE, 8), STRIPE)],
                        out_hbm.at[cid, pl.ds(pl.multiple_of(sid * STRIPE, 8),
                                              STRIPE)])

    return sc_pass


_sc_pass1 = _make_sc_pass(DIM, num_cores=2)


def _make_sc_pass0():
    d = HID
    H = d // 4
    RW = H + 16
    NCORES = 2
    tiles = 16 * NCORES
    cpt = NCH // tiles
    EPT = cpt * C
    inv = 1.0 / math.sqrt(d)
    mesh = plsc.VectorSubcoreMesh(core_axis_name="c", subcore_axis_name="s",
                                  num_cores=NCORES)
    cp = pltpu.CompilerParams()
    if "needs_layout_passes" in pltpu.CompilerParams.__dataclass_fields__:
        cp = dataclasses.replace(cp, needs_layout_passes=False,
                                 use_tc_tiling_on_sc=False)

    @functools.partial(
        pl.kernel,
        out_type=jax.ShapeDtypeStruct((4, NCORES, NACC, RW), jnp.float32),
        mesh=mesh,
        compiler_params=cp,
        scratch_types=[
            pltpu.VMEM((NP,), jnp.float32),
            pltpu.VMEM((C,), jnp.int32),
            pltpu.VMEM((C,), jnp.int32),
            pltpu.VMEM((C,), jnp.float32),
            pltpu.VMEM((C, d), jnp.float32),
            pltpu.VMEM((C, d), jnp.float32),
            pltpu.VMEM((C, H), jnp.float32),
            pltpu.VMEM((C, RW), jnp.float32),
            pltpu.VMEM((C,), jnp.float32),
            pltpu.VMEM((EPT + 16,), jnp.float32),
            pltpu.VMEM((EPT + 16,), jnp.float32),
            pltpu.VMEM_SHARED((NACC, RW), jnp.float32),
        ],
    )
    def sc_pass0(q_hbm, k_hbm, va_hbm, vb_hbm, vc_hbm, vd_hbm, u_hbm,
                 src_hbm, dst_hbm, a_hbm, z_hbm, out_hbm, u_v, src_v, dst_v,
                 a_v, q_v, k_v, v_v, r_v, s_v, ex_v, exa_v, acc_s):
        cid = lax.axis_index("c")
        sid = lax.axis_index("s")
        wid = sid * NCORES + cid

        pltpu.sync_copy(u_hbm, u_v)
        stripe = pl.ds(pl.multiple_of(sid * STRIPE, 8), STRIPE)
        pltpu.sync_copy(z_hbm, acc_s.at[stripe])
        plsc.subcore_barrier()

        i16 = lax.iota(jnp.int32, 16)
        m0 = i16 == 0
        m1 = i16 == 1

        @pl.loop(0, cpt)
        def _chunk(t):
            ch = wid * cpt + t
            off = t * C
            pltpu.sync_copy(src_hbm.at[ch], src_v)
            pltpu.sync_copy(dst_hbm.at[ch], dst_v)
            pltpu.sync_copy(a_hbm.at[ch], a_v)
            pltpu.sync_copy(k_hbm.at[src_v], k_v)
            pltpu.sync_copy(va_hbm.at[src_v], v_v)
            pltpu.sync_copy(q_hbm.at[dst_v], q_v)

            @pl.loop(0, C)
            def _dotloop(e):
                acc = q_v[e, pl.ds(0, 16)] * k_v[e, pl.ds(0, 16)]
                for j in range(1, d // 16):
                    acc = acc + q_v[e, pl.ds(16 * j, 16)] * k_v[e, pl.ds(16 * j, 16)]
                tot = jnp.sum(acc)
                base = (e // 16) * 16
                sl = pl.ds(base, 16)
                old = s_v[sl]
                s_v[sl] = jnp.where(i16 == (e - base), jnp.full((16,), tot),
                                    old)

            for g in range(C // 16):
                sl = pl.ds(g * 16, 16)
                osl = pl.ds(off + g * 16, 16)
                dst16 = dst_v[sl]
                u16 = plsc.load_gather(u_v, [dst16])
                a16 = a_v[sl]
                ex16 = jnp.exp((s_v[sl] + a16 * u16) * inv)
                ex_v[osl] = ex16
                exa_v[osl] = ex16 * a16

            @pl.loop(0, C)
            def _rowloop(e):
                ex_bc = jnp.full((16,), ex_v[pl.ds(off + e, 16)][0])
                exa_bc = jnp.full((16,), exa_v[pl.ds(off + e, 16)][0])
                r_v[e, pl.ds(H, 16)] = (jnp.where(m0, ex_bc, 0.0) +
                                        jnp.where(m1, exa_bc, 0.0))
                for j in range(H // 16):
                    r_v[e, pl.ds(16 * j, 16)] = v_v[e, pl.ds(16 * j, 16)] * ex_bc

            pltpu.sync_copy(r_v, acc_s.at[dst_v], add=True)

        plsc.subcore_barrier()
        pltpu.sync_copy(acc_s.at[stripe], out_hbm.at[0, cid, stripe])

        z16 = jnp.zeros((16,), jnp.float32)
        for p, vp_hbm in enumerate([vb_hbm, vc_hbm, vd_hbm], start=1):
            plsc.subcore_barrier()
            pltpu.sync_copy(z_hbm, acc_s.at[stripe])
            plsc.subcore_barrier()
            if p == 1:
                @pl.loop(0, C)
                def _zerotail(e):
                    r_v[e, pl.ds(H, 16)] = z16

            @pl.loop(0, cpt)
            def _chunk_b(t):
                ch = wid * cpt + t
                off = t * C
                pltpu.sync_copy(dst_hbm.at[ch], dst_v)
                pltpu.sync_copy(src_hbm.at[ch], src_v)
                pltpu.sync_copy(vp_hbm.at[src_v], v_v)

                @pl.loop(0, C)
                def _rowloop_b(e):
                    ex_bc = jnp.full((16,), ex_v[pl.ds(off + e, 16)][0])
                    for j in range(H // 16):
                        r_v[e, pl.ds(16 * j, 16)] = (
                            v_v[e, pl.ds(16 * j, 16)] * ex_bc)

                pltpu.sync_copy(r_v, acc_s.at[dst_v], add=True)

            plsc.subcore_barrier()
            pltpu.sync_copy(acc_s.at[stripe], out_hbm.at[p, cid, stripe])

    return sc_pass0


_sc_pass0 = _make_sc_pass0()


def _combine0_body(acc_ref, skip_ref, we0_ref, g0_ref, be0_ref, h0f_ref):
    H = HID // 4
    num = jnp.concatenate([acc_ref[p, 0:N, 0:H] for p in range(4)], axis=1)
    den = jnp.maximum(acc_ref[0, 0:N, H:H + 1], 1e-16)
    w = acc_ref[0, 0:N, H + 1:H + 2]
    out0 = num / den + (w / den) * we0_ref[...] + skip_ref[0:N, :]
    mu = jnp.mean(out0, axis=0, keepdims=True)
    var = jnp.mean(out0 * out0, axis=0, keepdims=True) - mu * mu
    h0 = (out0 - mu) / jnp.sqrt(var + 1e-5) * g0_ref[...][None, :] + \
        be0_ref[...][None, :]
    h0f_ref[0:N, :] = h0
    h0f_ref[N:NP, :] = jnp.zeros((NP - N, HID), jnp.float32)


def _combine0(acc, skip0, We0, g0, be0):
    return pl.pallas_call(
        _combine0_body,
        out_shape=jax.ShapeDtypeStruct((NP, HID), jnp.float32),
    )(acc, skip0, We0, g0, be0)


def _dense1_body(x_ref, wq_ref, bq_ref, wk_ref, bk_ref, wv_ref, bv_ref,
                 we_ref, ws_ref, bs_ref,
                 q_ref, k_ref, v_ref, s_ref, u_ref):
    x = x_ref[...]
    q = _dot(x, wq_ref[...]) + bq_ref[...][None, :]
    q_ref[...] = q
    k_ref[...] = _dot(x, wk_ref[...]) + bk_ref[...][None, :]
    v_ref[...] = _dot(x, wv_ref[...]) + bv_ref[...][None, :]
    s_ref[...] = _dot(x, ws_ref[...]) + bs_ref[...][None, :]
    u2 = lax.dot_general(q, we_ref[...], (((1,), (1,)), ((), ())),
                         precision=_HIGH, preferred_element_type=jnp.float32)
    u_ref[...] = u2[:, 0]


def _dense1(h0f, Wq, bq, Wk, bk, Wv, bv, We, Ws, bs):
    d = Wq.shape[1]
    blk = 1024
    grid = NP // blk
    full2 = lambda a: pl.BlockSpec(a.shape, lambda i: (0,) * a.ndim)
    row = pl.BlockSpec((blk, HID), lambda i: (i, 0))
    rowo = pl.BlockSpec((blk, d), lambda i: (i, 0))
    return pl.pallas_call(
        _dense1_body,
        grid=(grid,),
        in_specs=[row, full2(Wq), full2(bq), full2(Wk), full2(bk),
                  full2(Wv), full2(bv), full2(We), full2(Ws), full2(bs)],
        out_specs=[rowo, rowo, rowo, rowo,
                   pl.BlockSpec((blk,), lambda i: (i,))],
        out_shape=[jax.ShapeDtypeStruct((NP, d), jnp.float32)] * 4 +
                  [jax.ShapeDtypeStruct((NP,), jnp.float32)],
    )(h0f, Wq, bq, Wk, bk, Wv, bv, We, Ws, bs)


def _combine1_body(acc_ref, skip_ref, h0f_ref, batch_ref, we1_ref,
                   g1_ref, be1_ref, pool_ref, xs_ref):
    num = acc_ref[0, 0:N, 0:DIM] + acc_ref[1, 0:N, 0:DIM]
    den = jnp.maximum(acc_ref[0, 0:N, DIM:DIM + 1] +
                      acc_ref[1, 0:N, DIM:DIM + 1], 1e-16)
    w = acc_ref[0, 0:N, DIM + 1:DIM + 2] + acc_ref[1, 0:N, DIM + 1:DIM + 2]
    out1 = num / den + (w / den) * we1_ref[...] + skip_ref[0:N, :]
    mu = jnp.mean(out1, axis=0, keepdims=True)
    var = jnp.mean(out1 * out1, axis=0, keepdims=True) - mu * mu
    h1 = (out1 - mu) / jnp.sqrt(var + 1e-5) * g1_ref[...][None, :] + \
        be1_ref[...][None, :]

    xs_ref[:, 0:HID] = h0f_ref[0:N, :]
    xs_ref[:, HID:HID + DIM] = h1

    b = batch_ref[...]
    onehot = (lax.broadcasted_iota(jnp.int32, (G, NP), 0) ==
              b[None, :]).astype(jnp.float32)
    xs_pad = jnp.concatenate(
        [h0f_ref[...],
         jnp.concatenate([h1, jnp.zeros((NP - N, DIM), jnp.float32)], axis=0)],
        axis=1)
    sums = _dot(onehot, xs_pad)
    cnt = jnp.sum(onehot, axis=1, keepdims=True)
    pool_ref[...] = sums / jnp.maximum(cnt, 1.0)


def _combine1(acc, skip1, h0f, batch_pad, We1, g1, be1):
    return pl.pallas_call(
        _combine1_body,
        out_shape=[jax.ShapeDtypeStruct((G, HID + DIM), jnp.float32),
                   jax.ShapeDtypeStruct((N, HID + DIM), jnp.float32)],
    )(acc, skip1, h0f, batch_pad, We1, g1, be1)


def _stage_body(ei_ref, ea_ref, src_ref, dst_ref, a_ref):
    src_ref[0:NCH_RAW, :] = ei_ref[0, :].reshape(NCH_RAW, C)
    src_ref[NCH_RAW:NCH, :] = jnp.zeros((NCH - NCH_RAW, C), jnp.int32)
    dst_ref[0:NCH_RAW, :] = ei_ref[1, :].reshape(NCH_RAW, C)
    dst_ref[NCH_RAW:NCH, :] = jnp.full((NCH - NCH_RAW, C), N, jnp.int32)
    a_ref[0:NCH_RAW, :] = ea_ref[:, 0].reshape(NCH_RAW, C)
    a_ref[NCH_RAW:NCH, :] = jnp.zeros((NCH - NCH_RAW, C), jnp.float32)


def _stage_edges(edge_index, edge_attr):
    return pl.pallas_call(
        _stage_body,
        out_shape=[jax.ShapeDtypeStruct((NCH, C), jnp.int32),
                   jax.ShapeDtypeStruct((NCH, C), jnp.int32),
                   jax.ShapeDtypeStruct((NCH, C), jnp.float32)],
    )(edge_index, edge_attr)


def kernel(x, edge_index, edge_attr, batch, Wq0, bq0, Wk0, bk0, Wv0, bv0,
           We0, Ws0, bs0, g0, be0, Wq1, bq1, Wk1, bk1, Wv1, bv1, We1, Ws1,
           bs1, g1, be1):
    x_pad = jnp.pad(x, ((0, NP - N), (0, 0)))
    src = edge_index[0]
    dst = edge_index[1]
    a = edge_attr[:, 0]
    srcp = jnp.concatenate([src, jnp.zeros((EP - E,), jnp.int32)]).reshape(NCH, C)
    dstp = jnp.concatenate([dst, jnp.full((EP - E,), N, jnp.int32)]).reshape(NCH, C)
    ap = jnp.concatenate([a, jnp.zeros((EP - E,), jnp.float32)]).reshape(NCH, C)
    batch_pad = jnp.concatenate([batch, jnp.full((NP - N,), G, jnp.int32)])
    z0 = jnp.zeros((STRIPE, HID // 4 + 16), jnp.float32)
    z1 = jnp.zeros((STRIPE, DIM + 16), jnp.float32)

    q0, k0, va0, vb0, vc0, vd0, skip0, u0 = _dense0(
        x_pad, Wq0, bq0, Wk0, bk0, Wv0, bv0, We0, Ws0, bs0)
    acc0 = _sc_pass0(q0, k0, va0, vb0, vc0, vd0, u0, srcp, dstp, ap, z0)
    acc0 = acc0[:, 0] + acc0[:, 1]
    h0f = _combine0(acc0, skip0, We0, g0, be0)
    q1, k1, v1, skip1, u1 = _dense1(h0f, Wq1, bq1, Wk1, bk1, Wv1, bv1,
                                    We1, Ws1, bs1)

    acc1 = _sc_pass1(q1, k1, v1, u1, srcp, dstp, ap, z1)
    pooled, xs = _combine1(acc1, skip1, h0f, batch_pad, We1, g1, be1)
    return (pooled, xs)
